# baseline (speedup 1.0000x reference)
import functools

import jax
import jax.numpy as jnp
from jax import lax
from jax.experimental import pallas as pl
from jax.experimental.pallas import tpu as pltpu
from jax.experimental.pallas import tpu_sc as plsc

B, T, H, K = 4, 4096, 1024, 512
TC = 1024
NW = 32
TPW = T * B // NW
CPB = T // TPW
GRP = 64
DUMMY_ROW = B * K
OUT_ROWS = B * K + 8


def _scores_body(h_ref, w_ref, o_ref):
    h = h_ref[...].astype(jnp.bfloat16)
    w = w_ref[...].astype(jnp.bfloat16)
    o_ref[0, :] = jnp.dot(h, w.T, preferred_element_type=jnp.float32)[:, 0]


def _scores_call(hidden, W):
    flat = hidden.reshape(B * T, H)
    n_chunks = B * T // TC
    return pl.pallas_call(
        _scores_body,
        grid=(n_chunks,),
        in_specs=[
            pl.BlockSpec((TC, H), lambda c: (c, 0)),
            pl.BlockSpec((1, H), lambda c: (0, 0)),
        ],
        out_specs=pl.BlockSpec((1, TC), lambda c: (0, c)),
        out_shape=jax.ShapeDtypeStruct((1, B * T), jnp.float32),
    )(flat, W)


def _prefix_sum(x):
    n = x.shape[1]
    acc = x
    sh = 1
    while sh < n:
        pad = jnp.zeros((x.shape[0], sh), acc.dtype)
        acc = acc + jnp.concatenate([pad, acc[:, : n - sh]], axis=1)
        sh *= 2
    return acc - x


def _select_body(s_ref, enc_ref, meta_ref):
    s = s_ref[...]
    bits = lax.bitcast_convert_type(s, jnp.int32)
    key = jnp.where(bits < 0, bits ^ jnp.int32(0x7FFFFFFF), bits)
    int_min = jnp.int32(-(2 ** 31))

    def search(i, tpat):
        bit = jnp.int32(31) - i
        cand = tpat | jnp.left_shift(jnp.int32(1), bit)
        thr = cand ^ int_min
        cnt = jnp.sum((key >= thr).astype(jnp.int32), axis=1, keepdims=True)
        return jnp.where(cnt >= K, cand, tpat)

    tpat = lax.fori_loop(0, 32, search, jnp.zeros((B, 1), jnp.int32))
    thr = tpat ^ int_min
    gt = key > thr
    eq = key == thr
    n_gt = jnp.sum(gt.astype(jnp.int32), axis=1, keepdims=True)
    need = K - n_gt
    eq_rank = _prefix_sum(eq.astype(jnp.int32))
    sel = gt | (eq & (eq_rank < need))

    t_loc = lax.broadcasted_iota(jnp.int32, (B, T), 1)
    b_ids = lax.broadcasted_iota(jnp.int32, (B, T), 0)
    enc_ref[...] = jnp.where(sel, t_loc + b_ids * T, jnp.int32(-1))

    sel_f = sel.astype(jnp.float32)
    tc_id = lax.broadcasted_iota(jnp.int32, (T, CPB), 0) // TPW
    c_id = lax.broadcasted_iota(jnp.int32, (T, CPB), 1)
    e_mat = (tc_id == c_id).astype(jnp.float32)
    cnts = jax.lax.dot(sel_f, e_mat,
                       precision=lax.Precision.HIGHEST)
    lo = lax.broadcasted_iota(jnp.int32, (CPB, CPB), 0)
    hi = lax.broadcasted_iota(jnp.int32, (CPB, CPB), 1)
    tri = (lo < hi).astype(jnp.float32)
    pstart = jax.lax.dot(cnts, tri,
                         precision=lax.Precision.HIGHEST)
    b_off = lax.broadcasted_iota(jnp.int32, (B, CPB), 0) * K
    meta_ref[...] = pstart.astype(jnp.int32) + b_off


def _select_call(scores):
    return pl.pallas_call(
        _select_body,
        out_shape=[
            jax.ShapeDtypeStruct((B, T), jnp.int32),
            jax.ShapeDtypeStruct((B, CPB), jnp.int32),
        ],
    )(scores)


def _gather_body(enc_hbm, meta_hbm, table_hbm, out_hbm,
                 enc_v, meta_v, cidx, tbuf, obuf, rows, sem_g, sem_s):
    nc = 2
    w = lax.axis_index("s") * nc + lax.axis_index("c")
    pltpu.sync_copy(enc_hbm.at[pl.ds(w * TPW, TPW)], enc_v)
    pltpu.sync_copy(meta_hbm, meta_v)

    lanes = lax.iota(jnp.int32, 16)
    zero16 = jnp.zeros((16,), jnp.int32)
    v0 = meta_v[pl.ds(0, 16)]
    v1 = meta_v[pl.ds(16, 16)]
    c0 = jnp.where(lanes == w, v0, zero16)
    c1 = jnp.where(lanes + 16 == w, v1, zero16)
    pstart = jnp.sum(c0) + jnp.sum(c1)

    zero = jnp.zeros((16,), jnp.int32)
    for i in range((TPW + GRP + 15) // 16):
        cidx[pl.ds(i * 16, 16)] = zero

    def compact(i, cnt):
        v = enc_v[pl.ds(i * 16, 16)]
        m = v >= 0
        plsc.store_compressed(cidx.at[pl.ds(cnt, 16)], v, mask=m)
        return cnt + jnp.sum(m.astype(jnp.int32))

    cnt = lax.fori_loop(0, TPW // 16, compact, jnp.int32(0))

    for g in range(TPW // GRP):
        @pl.when(cnt > g * GRP)
        def _():
            for i in range(GRP // 16):
                tbuf[pl.ds(i * 16, 16)] = cidx[pl.ds(g * GRP + i * 16, 16)]
                lane_pos = g * GRP + i * 16 + lanes
                obuf[pl.ds(i * 16, 16)] = jnp.where(
                    lane_pos < cnt, pstart + lane_pos, jnp.int32(DUMMY_ROW))
            pltpu.async_copy(table_hbm.at[tbuf], rows, sem_g).wait()
            pltpu.async_copy(rows, out_hbm.at[obuf], sem_s).wait()


def _gather_call(enc_flat, meta_flat, table):
    mesh = plsc.VectorSubcoreMesh(core_axis_name="c", subcore_axis_name="s")
    fn = functools.partial(
        pl.kernel,
        out_type=jax.ShapeDtypeStruct((OUT_ROWS, H), jnp.float32),
        mesh=mesh,
        compiler_params=pltpu.CompilerParams(needs_layout_passes=False),
        scratch_types=[
            pltpu.VMEM((TPW,), jnp.int32),
            pltpu.VMEM((NW,), jnp.int32),
            pltpu.VMEM((TPW + GRP, ), jnp.int32),
            pltpu.VMEM((GRP,), jnp.int32),
            pltpu.VMEM((GRP,), jnp.int32),
            pltpu.VMEM((GRP, H), jnp.float32),
            pltpu.SemaphoreType.DMA,
            pltpu.SemaphoreType.DMA,
        ],
    )(_gather_body)
    return fn(enc_flat, meta_flat, table)


def kernel(hidden, W, b):
    del b
    scores = _scores_call(hidden, W).reshape(B, T)

    enc, meta = _select_call(scores)
    out = _gather_call(enc.reshape(B * T), meta.reshape(NW),
                       hidden.reshape(B * T, H))
    memory = out[: B * K].reshape(B, K, H)
    mask = jnp.ones((B, K), hidden.dtype)
    return memory, mask


# speedup vs baseline: 1.7978x; 1.7978x over previous
import functools

import jax
import jax.numpy as jnp
from jax import lax
from jax.experimental import pallas as pl
from jax.experimental.pallas import tpu as pltpu
from jax.experimental.pallas import tpu_sc as plsc

B, T, H, K = 4, 4096, 1024, 512
TC = 2048
NW = 32
TPW = T * B // NW
CPB = T // TPW
GRP = 32
NG_FAST = 4
OUT_ROWS = B * K


def _scores_body(h_ref, w_ref, o_ref):
    h = h_ref[...].astype(jnp.bfloat16)
    w = w_ref[...].astype(jnp.bfloat16)
    o_ref[0, :] = jnp.dot(h, w.T, preferred_element_type=jnp.float32)[:, 0]


def _scores_call(hidden, W):
    flat = hidden.reshape(B * T, H)
    n_chunks = B * T // TC
    return pl.pallas_call(
        _scores_body,
        grid=(n_chunks,),
        in_specs=[
            pl.BlockSpec((TC, H), lambda c: (c, 0)),
            pl.BlockSpec((1, H), lambda c: (0, 0)),
        ],
        out_specs=pl.BlockSpec((1, TC), lambda c: (0, c)),
        out_shape=jax.ShapeDtypeStruct((1, B * T), jnp.float32),
    )(flat, W)


def _prefix_sum(x):
    n = x.shape[1]
    acc = x
    sh = 1
    while sh < n:
        pad = jnp.zeros((x.shape[0], sh), acc.dtype)
        acc = acc + jnp.concatenate([pad, acc[:, : n - sh]], axis=1)
        sh *= 2
    return acc - x


def _select_body(s_ref, enc_ref, meta_ref):
    s = s_ref[...]
    bits = lax.bitcast_convert_type(s, jnp.int32)
    key = jnp.where(bits < 0, bits ^ jnp.int32(0x7FFFFFFF), bits)
    int_min = jnp.int32(-(2 ** 31))

    def search(i, tpat):
        bit = jnp.int32(31) - i
        cand = tpat | jnp.left_shift(jnp.int32(1), bit)
        thr = cand ^ int_min
        cnt = jnp.sum((key >= thr).astype(jnp.int32), axis=1, keepdims=True)
        return jnp.where(cnt >= K, cand, tpat)

    tpat = lax.fori_loop(0, 32, search, jnp.zeros((B, 1), jnp.int32))
    thr = tpat ^ int_min
    gt = key > thr
    eq = key == thr
    n_gt = jnp.sum(gt.astype(jnp.int32), axis=1, keepdims=True)
    need = K - n_gt
    eq_rank = _prefix_sum(eq.astype(jnp.int32))
    sel = gt | (eq & (eq_rank < need))

    t_loc = lax.broadcasted_iota(jnp.int32, (B, T), 1)
    b_ids = lax.broadcasted_iota(jnp.int32, (B, T), 0)
    enc_ref[...] = jnp.where(sel, t_loc + b_ids * T, jnp.int32(-1))

    sel_f = sel.astype(jnp.float32)
    tc_id = lax.broadcasted_iota(jnp.int32, (T, CPB), 0) // TPW
    c_id = lax.broadcasted_iota(jnp.int32, (T, CPB), 1)
    e_mat = (tc_id == c_id).astype(jnp.float32)
    cnts = jax.lax.dot(sel_f, e_mat,
                       precision=lax.Precision.HIGHEST)
    lo = lax.broadcasted_iota(jnp.int32, (CPB, CPB), 0)
    hi = lax.broadcasted_iota(jnp.int32, (CPB, CPB), 1)
    tri = (lo < hi).astype(jnp.float32)
    pstart = jax.lax.dot(cnts, tri,
                         precision=lax.Precision.HIGHEST)
    b_off = lax.broadcasted_iota(jnp.int32, (B, CPB), 0) * K
    meta_ref[...] = pstart.astype(jnp.int32) + b_off


def _select_call(scores):
    return pl.pallas_call(
        _select_body,
        out_shape=[
            jax.ShapeDtypeStruct((B, T), jnp.int32),
            jax.ShapeDtypeStruct((B, CPB), jnp.int32),
        ],
    )(scores)


def _gather_body(enc_hbm, meta_hbm, table_hbm, out_hbm,
                 enc_v, meta_v, cidx,
                 tbuf0, tbuf1, tbuf2, obuf0, obuf1, obuf2,
                 rows0, rows1, rows2,
                 sg0, sg1, sg2, ss0, ss1, ss2):
    nc = 2
    w = lax.axis_index("s") * nc + lax.axis_index("c")
    pltpu.sync_copy(enc_hbm.at[pl.ds(w * TPW, TPW)], enc_v)
    pltpu.sync_copy(meta_hbm, meta_v)

    lanes = lax.iota(jnp.int32, 16)
    zero16 = jnp.zeros((16,), jnp.int32)
    v0 = meta_v[pl.ds(0, 16)]
    v1 = meta_v[pl.ds(16, 16)]
    c0 = jnp.where(lanes == w, v0, zero16)
    c1 = jnp.where(lanes + 16 == w, v1, zero16)
    pstart = jnp.sum(c0) + jnp.sum(c1)

    for i in range((TPW + GRP + 15) // 16):
        cidx[pl.ds(i * 16, 16)] = zero16

    def compact(i, carry):
        cnt, rep = carry
        v = enc_v[pl.ds(i * 16, 16)]
        m = v >= 0
        plsc.store_compressed(cidx.at[pl.ds(cnt, 16)], v, mask=m)
        return cnt + jnp.sum(m.astype(jnp.int32)), jnp.maximum(rep, jnp.max(v))

    cnt, rep = lax.fori_loop(0, TPW // 16, compact,
                             (jnp.int32(0), jnp.int32(0)))

    bufs = [(tbuf0, obuf0, rows0, sg0, ss0), (tbuf1, obuf1, rows1, sg1, ss1),
            (tbuf2, obuf2, rows2, sg2, ss2)]
    nbuf = len(bufs)
    ng = NG_FAST

    def cond(j):
        return cnt > j * GRP

    def full(g):
        return cnt >= (g + 1) * GRP

    def issue_gather(g):
        tb, ob, rw, sg, ss = bufs[g % nbuf]
        for i in range(GRP // 16):
            v = cidx[pl.ds(g * GRP + i * 16, 16)]
            lane_pos = g * GRP + i * 16 + lanes
            ok = lane_pos < cnt
            tb[pl.ds(i * 16, 16)] = jnp.where(ok, v, zero16 + rep)
            ob[pl.ds(i * 16, 16)] = jnp.where(
                ok, pstart + lane_pos, pstart + cnt - 1)
        pltpu.async_copy(table_hbm.at[tb], rw, sg)

    def wait_gather(g):
        tb, ob, rw, sg, ss = bufs[g % nbuf]
        pltpu.make_async_copy(table_hbm.at[tb], rw, sg).wait()

    def issue_scatter(g):
        tb, ob, rw, sg, ss = bufs[g % nbuf]
        pltpu.async_copy(rw, out_hbm.at[ob], ss)

    def wait_scatter(g):
        tb, ob, rw, sg, ss = bufs[g % nbuf]
        pltpu.make_async_copy(rw, out_hbm.at[ob], ss).wait()

    for g in range(min(nbuf, ng)):
        @pl.when(cond(g))
        def _(g=g):
            issue_gather(g)

    for g in range(ng):
        @pl.when(cond(g))
        def _(g=g):
            wait_gather(g)
            issue_scatter(g)

        if g + nbuf < ng:
            @pl.when(cond(g + nbuf))
            def _(g=g):
                wait_scatter(g)
                issue_gather(g + nbuf)

    for g in range(ng):
        later = cond(g + nbuf) if g + nbuf < ng else False
        @pl.when(cond(g) & jnp.logical_not(later))
        def _(g=g):
            wait_scatter(g)

    tb, ob, rw, sg, ss = bufs[0]

    def slow_group(g, _):
        base = g * GRP
        for i in range(GRP // 16):
            v = cidx[pl.ds(base + i * 16, 16)]
            lane_pos = base + i * 16 + lanes
            ok = lane_pos < cnt
            tb[pl.ds(i * 16, 16)] = jnp.where(ok, v, zero16 + rep)
            ob[pl.ds(i * 16, 16)] = jnp.where(
                ok, pstart + lane_pos, pstart + cnt - 1)
        pltpu.async_copy(table_hbm.at[tb], rw, sg).wait()
        pltpu.async_copy(rw, out_hbm.at[ob], ss).wait()
        return 0

    ngroups = (cnt + GRP - 1) // GRP
    lax.fori_loop(NG_FAST, ngroups, slow_group, 0)


def _gather_call(enc_flat, meta_flat, table):
    mesh = plsc.VectorSubcoreMesh(core_axis_name="c", subcore_axis_name="s")
    fn = functools.partial(
        pl.kernel,
        out_type=jax.ShapeDtypeStruct((OUT_ROWS, H), jnp.float32),
        mesh=mesh,
        compiler_params=pltpu.CompilerParams(needs_layout_passes=False),
        scratch_types=[
            pltpu.VMEM((TPW,), jnp.int32),
            pltpu.VMEM((NW,), jnp.int32),
            pltpu.VMEM((TPW + GRP,), jnp.int32),
            pltpu.VMEM((GRP,), jnp.int32),
            pltpu.VMEM((GRP,), jnp.int32),
            pltpu.VMEM((GRP,), jnp.int32),
            pltpu.VMEM((GRP,), jnp.int32),
            pltpu.VMEM((GRP,), jnp.int32),
            pltpu.VMEM((GRP,), jnp.int32),
            pltpu.VMEM((GRP, H), jnp.float32),
            pltpu.VMEM((GRP, H), jnp.float32),
            pltpu.VMEM((GRP, H), jnp.float32),
            pltpu.SemaphoreType.DMA,
            pltpu.SemaphoreType.DMA,
            pltpu.SemaphoreType.DMA,
            pltpu.SemaphoreType.DMA,
            pltpu.SemaphoreType.DMA,
            pltpu.SemaphoreType.DMA,
        ],
    )(_gather_body)
    return fn(enc_flat, meta_flat, table)


def kernel(hidden, W, b):
    del b
    scores = _scores_call(hidden, W).reshape(B, T)

    enc, meta = _select_call(scores)
    out = _gather_call(enc.reshape(B * T), meta.reshape(NW),
                       hidden.reshape(B * T, H))
    memory = out.reshape(B, K, H)
    mask = jnp.ones((B, K), hidden.dtype)
    return memory, mask
